# baseline (speedup 1.0000x reference)
import functools

import jax
import jax.numpy as jnp
from jax import lax
from jax.experimental import pallas as pl
from jax.experimental.pallas import tpu as pltpu
from jax.experimental.pallas import tpu_sc as plsc

N_NODES = 50000
N_EDGES = 1600000
NUM_GRAPHS = 512
LANES = 16

NUM_CORES = 2
NUM_SUBCORES = 16
NW = NUM_CORES * NUM_SUBCORES

BLK = 128
TOTAL_BLOCKS = N_EDGES // BLK
BASE_BLOCKS = TOTAL_BLOCKS // NW
EXTRA = TOTAL_BLOCKS - BASE_BLOCKS * NW

CB = 39
CHUNK = CB * BLK
NFULL = BASE_BLOCKS // CB
NPAIR = NFULL // 2
UNROLL = 8


@functools.cache
def _build_occlusion_sc():
    mesh = plsc.VectorSubcoreMesh(
        core_axis_name="c", subcore_axis_name="s",
        num_cores=NUM_CORES, num_subcores=NUM_SUBCORES,
    )

    @functools.partial(
        pl.kernel,
        mesh=mesh,
        out_type=jax.ShapeDtypeStruct((NW, LANES), jnp.float32),
        compiler_params=pltpu.CompilerParams(needs_layout_passes=False),
        scratch_types=[
            pltpu.VMEM((N_NODES,), jnp.int32),
            pltpu.VMEM((2, CHUNK), jnp.int32),
            pltpu.VMEM((2, CHUNK), jnp.int32),
            pltpu.VMEM((LANES,), jnp.float32),
            pltpu.SemaphoreType.DMA,
            pltpu.SemaphoreType.DMA,
            pltpu.SemaphoreType.DMA,
        ],
    )
    def occlusion_sc(tab_hbm, edge_hbm, out_hbm,
                     tab_v, buf_a, buf_b, acc_v,
                     sem_t, sem_a, sem_b):
        wid = lax.axis_index("s") * NUM_CORES + lax.axis_index("c")
        start_blk = wid * BASE_BLOCKS + jnp.minimum(wid, EXTRA)
        has_extra = wid < EXTRA
        base = start_blk * BLK

        def edge_slice(c):
            return edge_hbm.at[:, pl.ds(base + c * CHUNK, CHUNK)]

        cp_t = pltpu.make_async_copy(tab_hbm, tab_v, sem_t)
        cp_t.start()
        pltpu.make_async_copy(edge_slice(0), buf_a, sem_a).start()
        cp_t.wait()

        hi_mask = jnp.int32(-65536)

        def edge_term(s, d):
            vs = plsc.load_gather(tab_v, [s])
            vd = plsc.load_gather(tab_v, [d])
            xs = plsc.bitcast(vs & hi_mask, jnp.float32)
            ys = plsc.bitcast(vs << 16, jnp.float32)
            xd = plsc.bitcast(vd & hi_mask, jnp.float32)
            yd = plsc.bitcast(vd << 16, jnp.float32)
            dx = xd - xs
            dy = yd - ys
            s2 = dx * dx + dy * dy
            bits = plsc.bitcast(s2, jnp.int32)
            seed = plsc.bitcast(
                jnp.int32(0x5F375A86) - (bits >> 1), jnp.float32)
            half = s2 * jnp.float32(0.5)
            y = seed * (jnp.float32(1.5) - half * seed * seed)
            dist = s2 * y
            return jnp.exp(-dist)

        def accumulate(buf, accs, n_edges):
            @plsc.parallel_loop(0, n_edges, UNROLL * LANES, unroll=1,
                                carry=accs)
            def body(o, accs):
                new = []
                for j in range(UNROLL):
                    s = buf[0, pl.ds(o + j * LANES, LANES)]
                    d = buf[1, pl.ds(o + j * LANES, LANES)]
                    new.append(accs[j] + edge_term(s, d))
                return tuple(new)

            return body

        def pair_body(p, accs):
            c0 = 2 * p
            pltpu.make_async_copy(edge_slice(c0 + 1), buf_b, sem_b).start()
            pltpu.make_async_copy(edge_slice(c0), buf_a, sem_a).wait()
            accs = accumulate(buf_a, accs, CHUNK)

            @pl.when(p + 1 < NPAIR)
            def _():
                pltpu.make_async_copy(edge_slice(c0 + 2), buf_a, sem_a).start()

            pltpu.make_async_copy(edge_slice(c0 + 1), buf_b, sem_b).wait()
            return accumulate(buf_b, accs, CHUNK)

        zeros = jnp.zeros((LANES,), jnp.float32)
        accs = lax.fori_loop(0, NPAIR, pair_body, (zeros,) * UNROLL)

        @pl.when(has_extra)
        def _():
            off = base + NFULL * CHUNK
            pltpu.sync_copy(edge_hbm.at[:, pl.ds(off, BLK)],
                            buf_a.at[:, pl.ds(0, BLK)])
            tail = accumulate(buf_a, accs, BLK)
            total = tail[0]
            for j in range(1, UNROLL):
                total = total + tail[j]
            acc_v[...] = total

        @pl.when(jnp.logical_not(has_extra))
        def _():
            total = accs[0]
            for j in range(1, UNROLL):
                total = total + accs[j]
            acc_v[...] = total

        pltpu.sync_copy(acc_v, out_hbm.at[wid])

    return occlusion_sc


def kernel(node_pos, full_edge_index, batch):
    del batch
    pos = node_pos.astype(jnp.float32)
    xu = lax.bitcast_convert_type(
        pos[:, 0].astype(jnp.bfloat16), jnp.uint16).astype(jnp.uint32)
    yu = lax.bitcast_convert_type(
        pos[:, 1].astype(jnp.bfloat16), jnp.uint16).astype(jnp.uint32)
    packed = lax.bitcast_convert_type((xu << 16) | yu, jnp.int32)
    edges = full_edge_index.astype(jnp.int32)
    partials = _build_occlusion_sc()(packed, edges)
    return jnp.sum(partials) / jnp.float32(NUM_GRAPHS)


# speedup vs baseline: 1.0193x; 1.0193x over previous
import functools

import jax
import jax.numpy as jnp
from jax import lax
from jax.experimental import pallas as pl
from jax.experimental.pallas import tpu as pltpu
from jax.experimental.pallas import tpu_sc as plsc

N_NODES = 50000
N_EDGES = 1600000
NUM_GRAPHS = 512
LANES = 16

NUM_CORES = 2
NUM_SUBCORES = 16
NW = NUM_CORES * NUM_SUBCORES

BLK = 128
TOTAL_BLOCKS = N_EDGES // BLK
BASE_BLOCKS = TOTAL_BLOCKS // NW
EXTRA = TOTAL_BLOCKS - BASE_BLOCKS * NW

CB = 39
CHUNK = CB * BLK
NFULL = BASE_BLOCKS // CB
NPAIR = NFULL // 2
UNROLL = 4


@functools.cache
def _build_occlusion_sc():
    mesh = plsc.VectorSubcoreMesh(
        core_axis_name="c", subcore_axis_name="s",
        num_cores=NUM_CORES, num_subcores=NUM_SUBCORES,
    )

    @functools.partial(
        pl.kernel,
        mesh=mesh,
        out_type=jax.ShapeDtypeStruct((NW, LANES), jnp.float32),
        compiler_params=pltpu.CompilerParams(needs_layout_passes=False),
        scratch_types=[
            pltpu.VMEM((N_NODES,), jnp.int32),
            pltpu.VMEM((2, CHUNK), jnp.int32),
            pltpu.VMEM((2, CHUNK), jnp.int32),
            pltpu.VMEM((LANES,), jnp.float32),
            pltpu.SemaphoreType.DMA,
            pltpu.SemaphoreType.DMA,
            pltpu.SemaphoreType.DMA,
        ],
    )
    def occlusion_sc(tab_hbm, edge_hbm, out_hbm,
                     tab_v, buf_a, buf_b, acc_v,
                     sem_t, sem_a, sem_b):
        wid = lax.axis_index("s") * NUM_CORES + lax.axis_index("c")
        start_blk = wid * BASE_BLOCKS + jnp.minimum(wid, EXTRA)
        has_extra = wid < EXTRA
        base = start_blk * BLK

        def edge_slice(c):
            return edge_hbm.at[:, pl.ds(base + c * CHUNK, CHUNK)]

        cp_t = pltpu.make_async_copy(tab_hbm, tab_v, sem_t)
        cp_t.start()
        pltpu.make_async_copy(edge_slice(0), buf_a, sem_a).start()
        cp_t.wait()

        hi_mask = jnp.int32(-65536)

        def edge_term(s, d):
            vs = plsc.load_gather(tab_v, [s])
            vd = plsc.load_gather(tab_v, [d])
            xs = plsc.bitcast(vs & hi_mask, jnp.float32)
            ys = plsc.bitcast(vs << 16, jnp.float32)
            xd = plsc.bitcast(vd & hi_mask, jnp.float32)
            yd = plsc.bitcast(vd << 16, jnp.float32)
            dx = xd - xs
            dy = yd - ys
            s2 = dx * dx + dy * dy
            bits = plsc.bitcast(s2, jnp.int32)
            seed = plsc.bitcast(
                jnp.int32(0x5F375A86) - (bits >> 1), jnp.float32)
            half = s2 * jnp.float32(0.5)
            y = seed * (jnp.float32(1.5) - half * seed * seed)
            dist = s2 * y
            return jnp.exp(-dist)

        def accumulate(buf, accs, n_edges):
            @plsc.parallel_loop(0, n_edges, UNROLL * LANES, unroll=2,
                                carry=accs)
            def body(o, accs):
                new = []
                for j in range(UNROLL):
                    s = buf[0, pl.ds(o + j * LANES, LANES)]
                    d = buf[1, pl.ds(o + j * LANES, LANES)]
                    new.append(accs[j] + edge_term(s, d))
                return tuple(new)

            return body

        def pair_body(p, accs):
            c0 = 2 * p
            pltpu.make_async_copy(edge_slice(c0 + 1), buf_b, sem_b).start()
            pltpu.make_async_copy(edge_slice(c0), buf_a, sem_a).wait()
            accs = accumulate(buf_a, accs, CHUNK)

            @pl.when(p + 1 < NPAIR)
            def _():
                pltpu.make_async_copy(edge_slice(c0 + 2), buf_a, sem_a).start()

            pltpu.make_async_copy(edge_slice(c0 + 1), buf_b, sem_b).wait()
            return accumulate(buf_b, accs, CHUNK)

        zeros = jnp.zeros((LANES,), jnp.float32)
        accs = lax.fori_loop(0, NPAIR, pair_body, (zeros,) * UNROLL)

        @pl.when(has_extra)
        def _():
            off = base + NFULL * CHUNK
            pltpu.sync_copy(edge_hbm.at[:, pl.ds(off, BLK)],
                            buf_a.at[:, pl.ds(0, BLK)])
            tail = accumulate(buf_a, accs, BLK)
            total = tail[0]
            for j in range(1, UNROLL):
                total = total + tail[j]
            acc_v[...] = total

        @pl.when(jnp.logical_not(has_extra))
        def _():
            total = accs[0]
            for j in range(1, UNROLL):
                total = total + accs[j]
            acc_v[...] = total

        pltpu.sync_copy(acc_v, out_hbm.at[wid])

    return occlusion_sc


def kernel(node_pos, full_edge_index, batch):
    del batch
    pos = node_pos.astype(jnp.float32)
    xu = lax.bitcast_convert_type(
        pos[:, 0].astype(jnp.bfloat16), jnp.uint16).astype(jnp.uint32)
    yu = lax.bitcast_convert_type(
        pos[:, 1].astype(jnp.bfloat16), jnp.uint16).astype(jnp.uint32)
    packed = lax.bitcast_convert_type((xu << 16) | yu, jnp.int32)
    edges = full_edge_index.astype(jnp.int32)
    partials = _build_occlusion_sc()(packed, edges)
    return jnp.sum(partials) / jnp.float32(NUM_GRAPHS)
